# Initial kernel scaffold; baseline (speedup 1.0000x reference)
import jax
import jax.numpy as jnp
from jax.experimental import pallas as pl


def kernel(histories, history_lengths, choice_sets, choice_set_lengths, Wh, Wc, Wt):
    raise NotImplementedError("write your pallas kernel here")


# speedup vs baseline: 1.4164x; 1.4164x over previous
import functools

import jax
import jax.numpy as jnp
from jax import lax
from jax.experimental import pallas as pl
from jax.experimental.pallas import tpu as pltpu
from jax.experimental.pallas import tpu_sc as plsc

_NUM_ITEMS = 1000000
_D = 16
_B = 16384
_H = 50
_C = 20
_BETA = 0.5

_NC = 2
_NS = 16
_NW = _NC * _NS
_RPW = _B // _NW


def _sc_body(hist_hbm, cs_hbm, wh_hbm, wc_hbm, wt_hbm, util_hbm,
             hidx_v, cidx_v, out_v,
             hr0, cr0, tr0, hr1, cr1, tr1,
             hs0, cs0, ts0, hs1, cs1, ts1):
    wid = lax.axis_index("s") * _NC + lax.axis_index("c")
    base = wid * _RPW

    pltpu.sync_copy(hist_hbm.at[pl.ds(base, _RPW)], hidx_v)
    pltpu.sync_copy(cs_hbm.at[pl.ds(base, _RPW)], cidx_v)

    bufs = ((hr0, cr0, tr0), (hr1, cr1, tr1))
    sems = ((hs0, cs0, ts0), (hs1, cs1, ts1))

    def issue(row, b):
        pltpu.async_copy(wh_hbm.at[hidx_v.at[row]], bufs[b][0], sems[b][0])
        pltpu.async_copy(wc_hbm.at[cidx_v.at[row]], bufs[b][1], sems[b][1])
        pltpu.async_copy(wt_hbm.at[cidx_v.at[row]], bufs[b][2], sems[b][2])

    def wait(row, b):
        pltpu.make_async_copy(
            wh_hbm.at[hidx_v.at[row]], bufs[b][0], sems[b][0]).wait()
        pltpu.make_async_copy(
            wc_hbm.at[cidx_v.at[row]], bufs[b][1], sems[b][1]).wait()
        pltpu.make_async_copy(
            wt_hbm.at[cidx_v.at[row]], bufs[b][2], sems[b][2]).wait()

    lanes = lax.iota(jnp.int32, _D)

    def compute(row, b):
        hr, cr, tr = bufs[b]
        acc = hr[0]
        for h in range(1, _H):
            acc = acc + hr[h] * (_BETA ** h)
        s = cr[0]
        for c in range(1, _C):
            s = s + cr[c]
        a = acc + s
        lo = jnp.zeros((_D,), jnp.float32)
        hi = jnp.zeros((_D,), jnp.float32)
        for c in range(_C):
            u = jnp.sum(tr[c] * (a - cr[c]))
            if c < _D:
                lo = jnp.where(lanes == c, u, lo)
            else:
                hi = jnp.where(lanes == (c - _D), u, hi)
        out_v[row, 0:_D] = lo
        out_v[row, _D:2 * _D] = hi

    issue(0, 0)

    def body(i, carry):
        r = i * 2
        for b in range(2):
            row = r + b
            nxt = row + 1

            @pl.when(nxt < _RPW)
            def _():
                issue(nxt, 1 - b)

            wait(row, b)
            compute(row, b)
        return carry

    lax.fori_loop(0, _RPW // 2, body, 0, unroll=False)

    pltpu.sync_copy(out_v, util_hbm.at[pl.ds(base, _RPW)])


_CP = 2 * _D

_sc_utilities = functools.partial(
    pl.kernel,
    out_type=jax.ShapeDtypeStruct((_B, _CP), jnp.float32),
    mesh=plsc.VectorSubcoreMesh(core_axis_name="c", subcore_axis_name="s"),
    compiler_params=pltpu.CompilerParams(
        needs_layout_passes=False, use_tc_tiling_on_sc=False),
    scratch_types=[
        pltpu.VMEM((_RPW, _H), jnp.int32),
        pltpu.VMEM((_RPW, _C), jnp.int32),
        pltpu.VMEM((_RPW, _CP), jnp.float32),
        pltpu.VMEM((_H, _D), jnp.float32),
        pltpu.VMEM((_C, _D), jnp.float32),
        pltpu.VMEM((_C, _D), jnp.float32),
        pltpu.VMEM((_H, _D), jnp.float32),
        pltpu.VMEM((_C, _D), jnp.float32),
        pltpu.VMEM((_C, _D), jnp.float32),
        pltpu.SemaphoreType.DMA,
        pltpu.SemaphoreType.DMA,
        pltpu.SemaphoreType.DMA,
        pltpu.SemaphoreType.DMA,
        pltpu.SemaphoreType.DMA,
        pltpu.SemaphoreType.DMA,
    ],
)(_sc_body)


def _softmax_body(u_ref, len_ref, o_ref):
    u = u_ref[...]
    ln = len_ref[...]
    col = lax.broadcasted_iota(jnp.int32, u.shape, 1)
    u = jnp.where((col >= ln) | (col >= _C), -jnp.inf, u)
    m = jnp.max(u, axis=1, keepdims=True)
    sh = u - m
    lse = jnp.log(jnp.sum(jnp.exp(sh), axis=1, keepdims=True))
    o_ref[...] = (sh - lse)[:, :_C]


_BLK = 2048


def _tc_logsoftmax(util, lens2d):
    return pl.pallas_call(
        _softmax_body,
        grid=(_B // _BLK,),
        in_specs=[
            pl.BlockSpec((_BLK, _CP), lambda i: (i, 0)),
            pl.BlockSpec((_BLK, 1), lambda i: (i, 0)),
        ],
        out_specs=pl.BlockSpec((_BLK, _C), lambda i: (i, 0)),
        out_shape=jax.ShapeDtypeStruct((_B, _C), jnp.float32),
    )(util, lens2d)


def kernel(histories, history_lengths, choice_sets, choice_set_lengths,
           Wh, Wc, Wt):
    del history_lengths
    util = _sc_utilities(histories, choice_sets, Wh, Wc, Wt)
    return _tc_logsoftmax(util, choice_set_lengths.reshape(_B, 1))
